# baseline (speedup 1.0000x reference)
import jax
import jax.numpy as jnp
from jax import lax
from jax.experimental import pallas as pl
from jax.experimental.pallas import tpu as pltpu
from jax.experimental.pallas import tpu_sc as plsc

NUM_VIEWS = 16
TOKEN_DIM = 256
BATCH = 16384
LANES = 16
NUM_CORES = 2
NUM_SUBCORES = 16
NUM_WORKERS = NUM_CORES * NUM_SUBCORES
ROWS_PER_WORKER = BATCH // NUM_WORKERS
CHUNK = 128
NUM_CHUNKS = ROWS_PER_WORKER // CHUNK
BLOCKS = CHUNK // LANES


def _lookup_body(idx_hbm, table_hbm, out_hbm, tab_v, idx_v,
                 buf0, buf1, ws0, ws1):
    wid = lax.axis_index("s") * NUM_CORES + lax.axis_index("c")
    base = wid * ROWS_PER_WORKER * TOKEN_DIM

    pltpu.sync_copy(table_hbm, tab_v)
    pltpu.sync_copy(idx_hbm.at[wid], idx_v)

    bufs = (buf0, buf1)
    wsems = (ws0, ws1)
    writes = [None, None]
    lane_rows = lax.iota(jnp.int32, LANES) * TOKEN_DIM
    dst_base = [lane_rows + rb * (LANES * TOKEN_DIM) for rb in range(BLOCKS)]

    for j in range(NUM_CHUNKS):
        buf = bufs[j % 2]
        if writes[j % 2] is not None:
            writes[j % 2].wait()

        src_base = [idx_v[pl.ds(j * CHUNK + rb * LANES, LANES)] * TOKEN_DIM
                    for rb in range(BLOCKS)]

        def col_body(c, _):
            csplat = jnp.full((LANES,), c, jnp.int32)
            for rb in range(BLOCKS):
                vals = plsc.load_gather(tab_v, [src_base[rb] + csplat])
                plsc.store_scatter(buf, [dst_base[rb] + csplat], vals)
            return 0

        lax.fori_loop(0, TOKEN_DIM, col_body, 0)
        writes[j % 2] = pltpu.async_copy(
            buf, out_hbm.at[pl.ds(base + j * CHUNK * TOKEN_DIM,
                                  CHUNK * TOKEN_DIM)], wsems[j % 2])

    writes[(NUM_CHUNKS - 2) % 2].wait()
    writes[(NUM_CHUNKS - 1) % 2].wait()


@jax.jit
def kernel(view_id, view_embed):
    idx = view_id.astype(jnp.int32).reshape(NUM_WORKERS, ROWS_PER_WORKER)
    run = pl.kernel(
        _lookup_body,
        out_type=jax.ShapeDtypeStruct((BATCH * TOKEN_DIM,), jnp.float32),
        mesh=plsc.VectorSubcoreMesh(core_axis_name="c", subcore_axis_name="s"),
        compiler_params=pltpu.CompilerParams(needs_layout_passes=False),
        scratch_types=[
            pltpu.VMEM((NUM_VIEWS * TOKEN_DIM,), jnp.float32),
            pltpu.VMEM((ROWS_PER_WORKER,), jnp.int32),
            pltpu.VMEM((CHUNK * TOKEN_DIM,), jnp.float32),
            pltpu.VMEM((CHUNK * TOKEN_DIM,), jnp.float32),
            pltpu.SemaphoreType.DMA,
            pltpu.SemaphoreType.DMA,
        ],
    )
    flat = run(idx, view_embed.reshape(-1))
    return flat.reshape(BATCH, TOKEN_DIM)


# speedup vs baseline: 4.9245x; 4.9245x over previous
import jax
import jax.numpy as jnp
from jax import lax
from jax.experimental import pallas as pl
from jax.experimental.pallas import tpu as pltpu
from jax.experimental.pallas import tpu_sc as plsc

NUM_VIEWS = 16
TOKEN_DIM = 256
BATCH = 16384
NUM_CORES = 2
NUM_SUBCORES = 16
NUM_WORKERS = NUM_CORES * NUM_SUBCORES
ROWS_PER_WORKER = BATCH // NUM_WORKERS
CHUNK = 128
NUM_CHUNKS = ROWS_PER_WORKER // CHUNK
NBUF = 3


def _gather_body(idx_hbm, table_hbm, out_hbm, idx_v,
                 buf0, buf1, buf2, gs0, gs1, gs2, ws0, ws1, ws2, ws3):
    wid = lax.axis_index("s") * NUM_CORES + lax.axis_index("c")
    base = wid * ROWS_PER_WORKER

    pltpu.sync_copy(idx_hbm.at[wid], idx_v)

    bufs = (buf0, buf1, buf2)
    gsems = (gs0, gs1, gs2)
    wsems = (ws0, ws1, ws2, ws3)
    gathers = [None] * NUM_CHUNKS
    writes = [None] * NUM_CHUNKS

    def start_gather(j):
        gathers[j] = pltpu.async_copy(
            table_hbm.at[idx_v.at[j]], bufs[j % NBUF], gsems[j % NBUF])

    for j in range(min(NBUF, NUM_CHUNKS)):
        start_gather(j)
    for j in range(NUM_CHUNKS):
        gathers[j].wait()
        writes[j] = pltpu.async_copy(
            bufs[j % NBUF], out_hbm.at[pl.ds(base + j * CHUNK, CHUNK)],
            wsems[j])
        nxt = j + NBUF
        if nxt < NUM_CHUNKS:
            writes[nxt - NBUF].wait()
            start_gather(nxt)
    for j in range(max(0, NUM_CHUNKS - NBUF), NUM_CHUNKS):
        writes[j].wait()


@jax.jit
def kernel(view_id, view_embed):
    idx = view_id.astype(jnp.int32).reshape(NUM_WORKERS, NUM_CHUNKS, CHUNK)
    table_rep = jnp.tile(view_embed, (NUM_WORKERS, 1))
    idx = idx + (jnp.arange(NUM_WORKERS, dtype=jnp.int32)
                 * NUM_VIEWS)[:, None, None]
    run = pl.kernel(
        _gather_body,
        out_type=jax.ShapeDtypeStruct((BATCH, TOKEN_DIM), jnp.float32),
        mesh=plsc.VectorSubcoreMesh(core_axis_name="c", subcore_axis_name="s"),
        scratch_types=[
            pltpu.VMEM((NUM_CHUNKS, CHUNK), jnp.int32),
            pltpu.VMEM((CHUNK, TOKEN_DIM), jnp.float32),
            pltpu.VMEM((CHUNK, TOKEN_DIM), jnp.float32),
            pltpu.VMEM((CHUNK, TOKEN_DIM), jnp.float32),
            pltpu.SemaphoreType.DMA,
            pltpu.SemaphoreType.DMA,
            pltpu.SemaphoreType.DMA,
            pltpu.SemaphoreType.DMA,
            pltpu.SemaphoreType.DMA,
            pltpu.SemaphoreType.DMA,
            pltpu.SemaphoreType.DMA,
        ],
    )
    return run(idx, table_rep)
